# Initial kernel scaffold; baseline (speedup 1.0000x reference)
import jax
import jax.numpy as jnp
from jax.experimental import pallas as pl


def kernel(x, edge_index, W1a, b1a, W1b, b1b, W2a, b2a, W2b, b2b, Wf, bf):
    raise NotImplementedError("write your pallas kernel here")


# speedup vs baseline: 8.4974x; 8.4974x over previous
import functools

import jax
import jax.numpy as jnp
from jax import lax
from jax.experimental import pallas as pl
from jax.experimental.pallas import tpu as pltpu
from jax.experimental.pallas import tpu_sc as plsc

_NC = 2
_NS = 16
_NW = _NC * _NS


def _make_segsum(N, H, E, EB, Np):
    e_per_w = E // _NW
    n_chunks = e_per_w // EB
    assert e_per_w * _NW == E and n_chunks * EB == e_per_w
    rows_per_tile = Np // _NS
    assert rows_per_tile * _NS == Np and rows_per_tile % 8 == 0 and Np >= N

    mesh = plsc.VectorSubcoreMesh(core_axis_name="c", subcore_axis_name="s")

    @functools.partial(
        pl.kernel,
        mesh=mesh,
        compiler_params=pltpu.CompilerParams(use_tc_tiling_on_sc=False),
        out_type=jax.ShapeDtypeStruct((_NC * Np, H), jnp.float32),
        scratch_types=[
            pltpu.VMEM((n_chunks, EB), jnp.int32),
            pltpu.VMEM((n_chunks, EB), jnp.int32),
            pltpu.VMEM((EB, H), jnp.float32),
            pltpu.VMEM_SHARED((Np, H), jnp.float32),
            pltpu.SemaphoreType.DMA,
        ],
    )
    def seg(y_hbm, src_hbm, dst_hbm, zero_hbm, out_hbm,
            src_v, dst_v, rows_v, acc_sh, sem):
        cid = lax.axis_index("c")
        sid = lax.axis_index("s")
        wid = sid * _NC + cid

        r0 = sid * rows_per_tile
        pltpu.sync_copy(zero_hbm.at[pl.ds(r0, rows_per_tile)],
                        acc_sh.at[pl.ds(r0, rows_per_tile)])

        pltpu.sync_copy(src_hbm.at[wid], src_v)
        pltpu.sync_copy(dst_hbm.at[wid], dst_v)
        plsc.subcore_barrier()

        def body(c, carry):
            pltpu.async_copy(y_hbm.at[src_v.at[c]], rows_v, sem).wait()
            pltpu.sync_copy(rows_v, acc_sh.at[dst_v.at[c]], add=True)
            return carry

        lax.fori_loop(0, n_chunks, body, 0, unroll=False)

        plsc.subcore_barrier()
        pltpu.sync_copy(acc_sh.at[pl.ds(r0, rows_per_tile)],
                        out_hbm.at[pl.ds(cid * Np + r0, rows_per_tile)])

    return seg


def _d1_body(x_ref, w_ref, o_ref):
    o_ref[...] = jnp.dot(x_ref[...], w_ref[...],
                         preferred_element_type=jnp.float32)


def _d2_body(y_ref, p_ref, b1a_ref, w1b_ref, b1b_ref, w2a_ref, o_ref):
    g = y_ref[...] + p_ref[0] + p_ref[1] + b1a_ref[...]
    r = jnp.maximum(g, 0.0)
    c1 = jnp.dot(r, w1b_ref[...], preferred_element_type=jnp.float32)
    h1 = jnp.maximum(c1 + b1b_ref[...], 0.0)
    o_ref[...] = jnp.dot(h1, w2a_ref[...], preferred_element_type=jnp.float32)


def _d3_body(y_ref, p_ref, b2a_ref, w2b_ref, b2b_ref, wf_ref, bf_ref, o_ref):
    g = y_ref[...] + p_ref[0] + p_ref[1] + b2a_ref[...]
    r = jnp.maximum(g, 0.0)
    c2 = jnp.dot(r, w2b_ref[...], preferred_element_type=jnp.float32)
    h2 = jnp.maximum(c2 + b2b_ref[...], 0.0)
    o = jnp.dot(h2, wf_ref[...], preferred_element_type=jnp.float32) + bf_ref[...]
    m = jnp.max(o, axis=1, keepdims=True)
    s = o - m
    lse = jnp.log(jnp.sum(jnp.exp(s), axis=1, keepdims=True))
    o_ref[...] = s - lse


def kernel(x, edge_index, W1a, b1a, W1b, b1b, W2a, b2a, W2b, b2b, Wf, bf):
    N, D = x.shape
    H = W1a.shape[0]
    C = Wf.shape[0]
    E = edge_index.shape[1]

    EB = 80
    e_per_w = E // _NW
    n_chunks = e_per_w // EB

    Np = ((N + 8 * _NS - 1) // (8 * _NS)) * (8 * _NS)
    srcr = edge_index[0].reshape(_NW, n_chunks, EB)
    dstr = edge_index[1].reshape(_NW, n_chunks, EB)
    zeros = jnp.zeros((Np, H), jnp.float32)
    segsum = _make_segsum(N, H, E, EB, Np)

    f32 = jnp.float32
    d1 = pl.pallas_call(_d1_body,
                        out_shape=jax.ShapeDtypeStruct((N, H), f32))
    d2 = pl.pallas_call(_d2_body,
                        out_shape=jax.ShapeDtypeStruct((N, H), f32))
    d3 = pl.pallas_call(_d3_body,
                        out_shape=jax.ShapeDtypeStruct((N, C), f32))

    y1 = d1(x, W1a.T)
    p1 = segsum(y1, srcr, dstr, zeros).reshape(_NC, Np, H)[:, :N, :]
    y2 = d2(y1, p1, b1a.reshape(1, H), W1b.T, b1b.reshape(1, H), W2a.T)
    p2 = segsum(y2, srcr, dstr, zeros).reshape(_NC, Np, H)[:, :N, :]
    out = d3(y2, p2, b2a.reshape(1, H), W2b.T, b2b.reshape(1, H), Wf.T,
             bf.reshape(1, C))
    return out


# speedup vs baseline: 17.8627x; 2.1021x over previous
import functools

import jax
import jax.numpy as jnp
from jax import lax
from jax.experimental import pallas as pl
from jax.experimental.pallas import tpu as pltpu
from jax.experimental.pallas import tpu_sc as plsc

_NC = 2
_NS = 16
_NW = _NC * _NS


def _make_segsum(N, H, E, EB, Np):
    e_per_w = E // _NW
    n_chunks = e_per_w // EB
    assert e_per_w * _NW == E and n_chunks * EB == e_per_w
    rows_per_tile = Np // _NS
    assert rows_per_tile * _NS == Np and rows_per_tile % 8 == 0 and Np >= N

    mesh = plsc.VectorSubcoreMesh(core_axis_name="c", subcore_axis_name="s")

    NBUF = 5
    assert n_chunks % NBUF == 0

    @functools.partial(
        pl.kernel,
        mesh=mesh,
        compiler_params=pltpu.CompilerParams(use_tc_tiling_on_sc=False),
        out_type=jax.ShapeDtypeStruct((_NC * Np, H), jnp.float32),
        scratch_types=[
            pltpu.VMEM((n_chunks, EB), jnp.int32),
            pltpu.VMEM((n_chunks, EB), jnp.int32),
            [pltpu.VMEM((EB, H), jnp.float32) for _ in range(NBUF)],
            pltpu.VMEM_SHARED((Np, H), jnp.float32),
            [pltpu.SemaphoreType.DMA for _ in range(NBUF)],
        ],
    )
    def seg(y_hbm, src_hbm, dst_hbm, zero_hbm, out_hbm,
            src_v, dst_v, rows_bufs, acc_sh, sems):
        cid = lax.axis_index("c")
        sid = lax.axis_index("s")
        wid = sid * _NC + cid

        idx_cp0 = pltpu.async_copy(src_hbm.at[wid], src_v, sems[0])
        idx_cp1 = pltpu.async_copy(dst_hbm.at[wid], dst_v, sems[1])

        r0 = sid * rows_per_tile
        pltpu.sync_copy(zero_hbm.at[pl.ds(r0, rows_per_tile)],
                        acc_sh.at[pl.ds(r0, rows_per_tile)])
        idx_cp0.wait()
        idx_cp1.wait()
        plsc.subcore_barrier()

        def gather(c, b):
            return pltpu.async_copy(y_hbm.at[src_v.at[c]], rows_bufs[b],
                                    sems[b])

        def gather_wait(c, b):
            pltpu.make_async_copy(y_hbm.at[src_v.at[c]], rows_bufs[b],
                                  sems[b]).wait()

        for b in range(NBUF - 1):
            gather(b, b)

        def body(i, carry):
            c0 = i * NBUF
            for b in range(NBUF):
                c = c0 + b
                nxt = c + NBUF - 1

                @pl.when(nxt < n_chunks)
                def _():
                    gather(nxt, (b + NBUF - 1) % NBUF)

                gather_wait(c, b)
                pltpu.sync_copy(rows_bufs[b], acc_sh.at[dst_v.at[c]],
                                add=True)
            return carry

        lax.fori_loop(0, n_chunks // NBUF, body, 0, unroll=False)

        plsc.subcore_barrier()
        pltpu.sync_copy(acc_sh.at[pl.ds(r0, rows_per_tile)],
                        out_hbm.at[pl.ds(cid * Np + r0, rows_per_tile)])

    return seg


def _d1_body(x_ref, w_ref, o_ref):
    o_ref[...] = jnp.dot(x_ref[...], w_ref[...],
                         preferred_element_type=jnp.float32)


def _d2_body(y_ref, p_ref, b1a_ref, w1b_ref, b1b_ref, w2a_ref, o_ref):
    g = y_ref[...] + p_ref[0] + p_ref[1] + b1a_ref[...]
    r = jnp.maximum(g, 0.0)
    c1 = jnp.dot(r, w1b_ref[...], preferred_element_type=jnp.float32)
    h1 = jnp.maximum(c1 + b1b_ref[...], 0.0)
    o_ref[...] = jnp.dot(h1, w2a_ref[...], preferred_element_type=jnp.float32)


def _d3_body(y_ref, p_ref, b2a_ref, w2b_ref, b2b_ref, wf_ref, bf_ref, o_ref):
    g = y_ref[...] + p_ref[0] + p_ref[1] + b2a_ref[...]
    r = jnp.maximum(g, 0.0)
    c2 = jnp.dot(r, w2b_ref[...], preferred_element_type=jnp.float32)
    h2 = jnp.maximum(c2 + b2b_ref[...], 0.0)
    o = jnp.dot(h2, wf_ref[...], preferred_element_type=jnp.float32) + bf_ref[...]
    m = jnp.max(o, axis=1, keepdims=True)
    s = o - m
    lse = jnp.log(jnp.sum(jnp.exp(s), axis=1, keepdims=True))
    o_ref[...] = s - lse


def kernel(x, edge_index, W1a, b1a, W1b, b1b, W2a, b2a, W2b, b2b, Wf, bf):
    N, D = x.shape
    H = W1a.shape[0]
    C = Wf.shape[0]
    E = edge_index.shape[1]

    EB = 80
    e_per_w = E // _NW
    n_chunks = e_per_w // EB

    Np = ((N + 8 * _NS - 1) // (8 * _NS)) * (8 * _NS)
    srcr = edge_index[0].reshape(_NW, n_chunks, EB)
    dstr = edge_index[1].reshape(_NW, n_chunks, EB)
    zeros = jnp.zeros((Np, H), jnp.float32)
    segsum = _make_segsum(N, H, E, EB, Np)

    f32 = jnp.float32
    d1 = pl.pallas_call(_d1_body,
                        out_shape=jax.ShapeDtypeStruct((N, H), f32))
    d2 = pl.pallas_call(_d2_body,
                        out_shape=jax.ShapeDtypeStruct((N, H), f32))
    d3 = pl.pallas_call(_d3_body,
                        out_shape=jax.ShapeDtypeStruct((N, C), f32))

    y1 = d1(x, W1a.T)
    p1 = segsum(y1, srcr, dstr, zeros).reshape(_NC, Np, H)[:, :N, :]
    y2 = d2(y1, p1, b1a.reshape(1, H), W1b.T, b1b.reshape(1, H), W2a.T)
    p2 = segsum(y2, srcr, dstr, zeros).reshape(_NC, Np, H)[:, :N, :]
    out = d3(y2, p2, b2a.reshape(1, H), W2b.T, b2b.reshape(1, H), Wf.T,
             bf.reshape(1, C))
    return out
